# baseline (speedup 1.0000x reference)
import jax
import jax.numpy as jnp
from jax import lax
from jax.experimental import pallas as pl
from jax.experimental.pallas import tpu as pltpu
from jax.experimental.pallas import tpu_sc as plsc

_N = 256
_B = 1024
_OUT = _N * (_N + 1) // 2
_NC = 2
_NS = 16
_NW = _NC * _NS
_BPW = _B // _NW
_BLK = 32
_NBLK = _N // _BLK
_COL0 = [128 if _BLK * q >= 128 else 0 for q in range(_NBLK)]

_OFF = [i * _N - (i * (i - 1)) // 2 for i in range(_N)]


def _copy_rows(outbuf, stage, lo, hi, col0):
    for i in range(hi - 1, lo - 1, -1):
        seg_len = _N - i
        nch = (seg_len + 15) // 16
        for k in range(1, nch + 1):
            col = _N - 16 * k
            dst = _OFF[i] + seg_len - 16 * k
            outbuf[pl.ds(dst, 16)] = stage[i - lo, pl.ds(col - col0, 16)]


def _body(x_hbm, out_hbm, *refs):
    stages = refs[:_NBLK]
    out0, out1 = refs[_NBLK], refs[_NBLK + 1]
    isems = refs[_NBLK + 2:2 * _NBLK + 2]
    sem_o0, sem_o1 = refs[2 * _NBLK + 2], refs[2 * _NBLK + 3]
    wid = lax.axis_index("s") * _NC + lax.axis_index("c")
    outbufs = (out0, out1)
    osems = (sem_o0, sem_o1)

    def step(t, carry):
        for p in range(2):
            b = wid * _BPW + 2 * t + p
            obuf, osem = outbufs[p], osems[p]
            cps = [None] * _NBLK
            for q in range(_NBLK - 1, -1, -1):
                cps[q] = pltpu.async_copy(
                    x_hbm.at[b, pl.ds(_BLK * q, _BLK),
                             pl.ds(_COL0[q], _N - _COL0[q])],
                    stages[q], isems[q])
            @pl.when(t > 0)
            def _():
                pltpu.make_async_copy(obuf, out_hbm.at[b], osem).wait()
            for q in range(_NBLK - 1, -1, -1):
                cps[q].wait()
                _copy_rows(obuf, stages[q], _BLK * q, _BLK * (q + 1), _COL0[q])
            pltpu.async_copy(obuf, out_hbm.at[b], osem)
        return carry

    lax.fori_loop(0, _BPW // 2, step, 0)
    pltpu.make_async_copy(out0, out_hbm.at[0], sem_o0).wait()
    pltpu.make_async_copy(out1, out_hbm.at[0], sem_o1).wait()


@jax.jit
def _run(x):
    f = pl.kernel(
        _body,
        out_type=jax.ShapeDtypeStruct((_B, _OUT), jnp.float32),
        mesh=plsc.VectorSubcoreMesh(core_axis_name="c", subcore_axis_name="s"),
        scratch_types=(
            [pltpu.VMEM((_BLK, _N - _COL0[q]), jnp.float32)
             for q in range(_NBLK)]
            + [pltpu.VMEM((_OUT,), jnp.float32),
               pltpu.VMEM((_OUT,), jnp.float32)]
            + [pltpu.SemaphoreType.DMA] * (_NBLK + 2)
        ),
    )
    return f(x)


def kernel(input):
    return _run(input)


# speedup vs baseline: 1.1612x; 1.1612x over previous
import jax
import jax.numpy as jnp
from jax import lax
from jax.experimental import pallas as pl
from jax.experimental.pallas import tpu as pltpu
from jax.experimental.pallas import tpu_sc as plsc

_N = 256
_H = 128
_B = 1024
_OUT = _N * (_N + 1) // 2
_NC = 2
_NS = 16
_NW = _NC * _NS
_BPW = _B // _NW

_OFF = [i * _N - (i * (i - 1)) // 2 for i in range(_N)]


def _phase1_heads(obuf, stage, lo, hi, col0):
    for i in range(hi - 1, lo - 1, -1):
        u = i & 15
        obuf[pl.ds(_OFF[i] - u, 16)] = stage[i - lo, pl.ds(i - u - col0, 16)]


def _phase2_interior(obuf, stage, lo, hi, col0, kmax):
    for k in range(1, kmax + 1):
        i1 = min(hi, _N - 16 * k)
        if i1 <= lo:
            continue
        col = _N - 16 * k
        scol = col - col0

        @plsc.parallel_loop(lo, i1, unroll=8)
        def _(i):
            off = i * _N - ((i * i - i) >> 1)
            obuf[pl.ds(off + col - i, 16)] = stage[i - lo, pl.ds(scol, 16)]


def _body(x_hbm, out_hbm, stage_lo, stage_hi, out0, out1,
          sem_lo, sem_hi, sem_o0, sem_o1):
    wid = lax.axis_index("s") * _NC + lax.axis_index("c")
    outbufs = (out0, out1)
    osems = (sem_o0, sem_o1)

    def step(t, carry):
        for p in range(2):
            b = wid * _BPW + 2 * t + p
            obuf, osem = outbufs[p], osems[p]
            cp_hi = pltpu.async_copy(
                x_hbm.at[b, pl.ds(_H, _H), pl.ds(_H, _H)], stage_hi, sem_hi)
            cp_lo = pltpu.async_copy(
                x_hbm.at[b, pl.ds(0, _H), pl.ds(0, _N)], stage_lo, sem_lo)
            @pl.when(t > 0)
            def _():
                pltpu.make_async_copy(obuf, out_hbm.at[b], osem).wait()
            cp_hi.wait()
            _phase1_heads(obuf, stage_hi, _H, _N, _H)
            _phase2_interior(obuf, stage_hi, _H, _N, _H, 7)
            cp_lo.wait()
            _phase1_heads(obuf, stage_lo, 0, _H, 0)
            _phase2_interior(obuf, stage_lo, 0, _H, 0, 15)
            pltpu.async_copy(obuf, out_hbm.at[b], osem)
        return carry

    lax.fori_loop(0, _BPW // 2, step, 0)
    pltpu.make_async_copy(out0, out_hbm.at[0], sem_o0).wait()
    pltpu.make_async_copy(out1, out_hbm.at[0], sem_o1).wait()


@jax.jit
def _run(x):
    f = pl.kernel(
        _body,
        out_type=jax.ShapeDtypeStruct((_B, _OUT), jnp.float32),
        mesh=plsc.VectorSubcoreMesh(core_axis_name="c", subcore_axis_name="s"),
        scratch_types=[
            pltpu.VMEM((_H, _N), jnp.float32),
            pltpu.VMEM((_H, _H), jnp.float32),
            pltpu.VMEM((_OUT,), jnp.float32),
            pltpu.VMEM((_OUT,), jnp.float32),
            pltpu.SemaphoreType.DMA,
            pltpu.SemaphoreType.DMA,
            pltpu.SemaphoreType.DMA,
            pltpu.SemaphoreType.DMA,
        ],
    )
    return f(x)


def kernel(input):
    return _run(input)


# speedup vs baseline: 1.1693x; 1.0070x over previous
import jax
import jax.numpy as jnp
from jax import lax
from jax.experimental import pallas as pl
from jax.experimental.pallas import tpu as pltpu
from jax.experimental.pallas import tpu_sc as plsc

_N = 256
_H = 128
_B = 1024
_OUT = _N * (_N + 1) // 2
_NC = 2
_NS = 16
_NW = _NC * _NS
_BPW = _B // _NW

_OFF = [i * _N - (i * (i - 1)) // 2 for i in range(_N)]


def _phase1_heads(obuf, stage, lo, hi, col0):
    for i in range(hi - 1, lo - 1, -1):
        u = i & 15
        obuf[pl.ds(_OFF[i] - u, 16)] = stage[i - lo, pl.ds(i - u - col0, 16)]


def _phase2_interior(obuf, stage, lo, hi, col0, kmax):
    for k in range(1, kmax + 1):
        i1 = min(hi, _N - 16 * k)
        if i1 <= lo:
            continue
        col = _N - 16 * k
        scol = col - col0

        @plsc.parallel_loop(lo, i1, unroll=8)
        def _(i):
            off = i * _N - ((i * i - i) >> 1)
            obuf[pl.ds(off + col - i, 16)] = stage[i - lo, pl.ds(scol, 16)]


def _issue_in(x_hbm, stage_lo, stage_hi, sem_lo, sem_hi, b):
    pltpu.async_copy(
        x_hbm.at[b, pl.ds(_H, _H), pl.ds(_H, _H)], stage_hi, sem_hi)
    pltpu.async_copy(
        x_hbm.at[b, pl.ds(0, _H), pl.ds(0, _N)], stage_lo, sem_lo)


def _body(x_hbm, out_hbm, stage_lo, stage_hi, out0, out1,
          sem_lo, sem_hi, sem_o0, sem_o1):
    wid = lax.axis_index("s") * _NC + lax.axis_index("c")
    outbufs = (out0, out1)
    osems = (sem_o0, sem_o1)
    first = wid * _BPW
    _issue_in(x_hbm, stage_lo, stage_hi, sem_lo, sem_hi, first)

    def step(t, carry):
        for p in range(2):
            b = first + 2 * t + p
            obuf, osem = outbufs[p], osems[p]
            pltpu.make_async_copy(
                x_hbm.at[b, pl.ds(_H, _H), pl.ds(_H, _H)], stage_hi,
                sem_hi).wait()
            @pl.when(t > 0)
            def _():
                pltpu.make_async_copy(obuf, out_hbm.at[b], osem).wait()
            _phase1_heads(obuf, stage_hi, _H, _N, _H)
            _phase2_interior(obuf, stage_hi, _H, _N, _H, 7)
            pltpu.make_async_copy(
                x_hbm.at[b, pl.ds(0, _H), pl.ds(0, _N)], stage_lo,
                sem_lo).wait()
            _phase1_heads(obuf, stage_lo, 0, _H, 0)
            _phase2_interior(obuf, stage_lo, 0, _H, 0, 15)
            if p == 0:
                _issue_in(x_hbm, stage_lo, stage_hi, sem_lo, sem_hi, b + 1)
            else:
                @pl.when(t < _BPW // 2 - 1)
                def _():
                    _issue_in(x_hbm, stage_lo, stage_hi, sem_lo, sem_hi,
                              b + 1)
            pltpu.async_copy(obuf, out_hbm.at[b], osem)
        return carry

    lax.fori_loop(0, _BPW // 2, step, 0)
    pltpu.make_async_copy(out0, out_hbm.at[0], sem_o0).wait()
    pltpu.make_async_copy(out1, out_hbm.at[0], sem_o1).wait()


@jax.jit
def _run(x):
    f = pl.kernel(
        _body,
        out_type=jax.ShapeDtypeStruct((_B, _OUT), jnp.float32),
        mesh=plsc.VectorSubcoreMesh(core_axis_name="c", subcore_axis_name="s"),
        scratch_types=[
            pltpu.VMEM((_H, _N), jnp.float32),
            pltpu.VMEM((_H, _H), jnp.float32),
            pltpu.VMEM((_OUT,), jnp.float32),
            pltpu.VMEM((_OUT,), jnp.float32),
            pltpu.SemaphoreType.DMA,
            pltpu.SemaphoreType.DMA,
            pltpu.SemaphoreType.DMA,
            pltpu.SemaphoreType.DMA,
        ],
    )
    return f(x)


def kernel(input):
    return _run(input)


# speedup vs baseline: 1.3872x; 1.1863x over previous
import jax
import jax.numpy as jnp
from jax import lax
from jax.experimental import pallas as pl
from jax.experimental.pallas import tpu as pltpu
from jax.experimental.pallas import tpu_sc as plsc

_N = 256
_H = 128
_B = 1024
_OUT = _N * (_N + 1) // 2
_NC = 2
_NS = 16
_NW = _NC * _NS
_BPW = _B // _NW

_OFF = [i * _N - (i * (i - 1)) // 2 for i in range(_N)]


def _phase1_heads(obuf, stage, lo, hi, col0):
    for i in range(hi - 1, lo - 1, -1):
        u = i & 15
        obuf[pl.ds(_OFF[i] - u, 16)] = stage[i - lo, pl.ds(i - u - col0, 16)]


def _phase2_interior(obuf, stage, lo, hi, col0, kmax):
    for k in range(1, kmax + 1):
        i1 = min(hi, _N - 16 * k)
        if i1 <= lo:
            continue
        col = _N - 16 * k
        scol = col - col0
        dst0 = jnp.int32(_OFF[lo] + col - lo)
        delta0 = jnp.int32(_N - 1 - lo)

        @plsc.parallel_loop(lo, i1, unroll=8, carry=(dst0, delta0))
        def _(i, c):
            dst, delta = c
            obuf[pl.ds(dst, 16)] = stage[i - lo, pl.ds(scol, 16)]
            return (dst + delta, delta - 1)


def _issue_in(x_hbm, stage_lo, stage_hi, sem_lo, sem_hi, b):
    pltpu.async_copy(
        x_hbm.at[b, pl.ds(_H, _H), pl.ds(_H, _H)], stage_hi, sem_hi)
    pltpu.async_copy(
        x_hbm.at[b, pl.ds(0, _H), pl.ds(0, _N)], stage_lo, sem_lo)


def _body(x_hbm, out_hbm, stage_lo, stage_hi, out0, out1,
          sem_lo, sem_hi, sem_o0, sem_o1):
    wid = lax.axis_index("s") * _NC + lax.axis_index("c")
    outbufs = (out0, out1)
    osems = (sem_o0, sem_o1)
    first = wid * _BPW
    _issue_in(x_hbm, stage_lo, stage_hi, sem_lo, sem_hi, first)

    def step(t, carry):
        for p in range(2):
            b = first + 2 * t + p
            obuf, osem = outbufs[p], osems[p]
            pltpu.make_async_copy(
                x_hbm.at[b, pl.ds(_H, _H), pl.ds(_H, _H)], stage_hi,
                sem_hi).wait()
            @pl.when(t > 0)
            def _():
                pltpu.make_async_copy(obuf, out_hbm.at[b], osem).wait()
            _phase1_heads(obuf, stage_hi, _H, _N, _H)
            _phase2_interior(obuf, stage_hi, _H, _N, _H, 7)
            pltpu.make_async_copy(
                x_hbm.at[b, pl.ds(0, _H), pl.ds(0, _N)], stage_lo,
                sem_lo).wait()
            _phase1_heads(obuf, stage_lo, 0, _H, 0)
            _phase2_interior(obuf, stage_lo, 0, _H, 0, 15)
            if p == 0:
                _issue_in(x_hbm, stage_lo, stage_hi, sem_lo, sem_hi, b + 1)
            else:
                @pl.when(t < _BPW // 2 - 1)
                def _():
                    _issue_in(x_hbm, stage_lo, stage_hi, sem_lo, sem_hi,
                              b + 1)
            pltpu.async_copy(obuf, out_hbm.at[b], osem)
        return carry

    lax.fori_loop(0, _BPW // 2, step, 0)
    pltpu.make_async_copy(out0, out_hbm.at[0], sem_o0).wait()
    pltpu.make_async_copy(out1, out_hbm.at[0], sem_o1).wait()


@jax.jit
def _run(x):
    f = pl.kernel(
        _body,
        out_type=jax.ShapeDtypeStruct((_B, _OUT), jnp.float32),
        mesh=plsc.VectorSubcoreMesh(core_axis_name="c", subcore_axis_name="s"),
        scratch_types=[
            pltpu.VMEM((_H, _N), jnp.float32),
            pltpu.VMEM((_H, _H), jnp.float32),
            pltpu.VMEM((_OUT,), jnp.float32),
            pltpu.VMEM((_OUT,), jnp.float32),
            pltpu.SemaphoreType.DMA,
            pltpu.SemaphoreType.DMA,
            pltpu.SemaphoreType.DMA,
            pltpu.SemaphoreType.DMA,
        ],
    )
    return f(x)


def kernel(input):
    return _run(input)


# speedup vs baseline: 1.4048x; 1.0127x over previous
import jax
import jax.numpy as jnp
from jax import lax
from jax.experimental import pallas as pl
from jax.experimental.pallas import tpu as pltpu
from jax.experimental.pallas import tpu_sc as plsc

_N = 256
_H = 128
_B = 1024
_OUT = _N * (_N + 1) // 2
_NC = 2
_NS = 16
_NW = _NC * _NS
_BPW = _B // _NW

_OFF = [i * _N - (i * (i - 1)) // 2 for i in range(_N)]


def _phase1_heads(obuf, stage, lo, hi, col0):
    for i in range(hi - 1, lo - 1, -1):
        u = i & 15
        obuf[pl.ds(_OFF[i] - u, 16)] = stage[i - lo, pl.ds(i - u - col0, 16)]


def _phase2_interior(obuf, stage, lo, hi, col0, kmax):
    for k in range(1, kmax + 1):
        i1 = min(hi, _N - 16 * k)
        if i1 <= lo:
            continue
        col = _N - 16 * k
        scol = col - col0
        dst0 = jnp.int32(_OFF[lo] + col - lo)
        delta0 = jnp.int32(_N - 1 - lo)

        @plsc.parallel_loop(lo, i1, unroll=16, carry=(dst0, delta0))
        def _(i, c):
            dst, delta = c
            obuf[pl.ds(dst, 16)] = stage[i - lo, pl.ds(scol, 16)]
            return (dst + delta, delta - 1)


def _issue_hi(x_hbm, stage_hi, sem_hi, b):
    pltpu.async_copy(
        x_hbm.at[b, pl.ds(_H, _H), pl.ds(_H, _H)], stage_hi, sem_hi)


def _issue_lo(x_hbm, stage_lo, sem_lo, b):
    pltpu.async_copy(
        x_hbm.at[b, pl.ds(0, _H), pl.ds(0, _N)], stage_lo, sem_lo)


def _body(x_hbm, out_hbm, stage_lo, stage_hi, out0, out1,
          sem_lo, sem_hi, sem_o0, sem_o1):
    wid = lax.axis_index("s") * _NC + lax.axis_index("c")
    outbufs = (out0, out1)
    osems = (sem_o0, sem_o1)
    first = wid * _BPW
    _issue_hi(x_hbm, stage_hi, sem_hi, first)
    _issue_lo(x_hbm, stage_lo, sem_lo, first)

    def step(t, carry):
        for p in range(2):
            b = first + 2 * t + p
            obuf, osem = outbufs[p], osems[p]
            pltpu.make_async_copy(
                x_hbm.at[b, pl.ds(_H, _H), pl.ds(_H, _H)], stage_hi,
                sem_hi).wait()
            @pl.when(t > 0)
            def _():
                pltpu.make_async_copy(obuf, out_hbm.at[b], osem).wait()
            _phase1_heads(obuf, stage_hi, _H, _N, _H)
            _phase2_interior(obuf, stage_hi, _H, _N, _H, 7)
            if p == 0:
                _issue_hi(x_hbm, stage_hi, sem_hi, b + 1)
            else:
                @pl.when(t < _BPW // 2 - 1)
                def _():
                    _issue_hi(x_hbm, stage_hi, sem_hi, b + 1)
            pltpu.make_async_copy(
                x_hbm.at[b, pl.ds(0, _H), pl.ds(0, _N)], stage_lo,
                sem_lo).wait()
            _phase1_heads(obuf, stage_lo, 0, _H, 0)
            _phase2_interior(obuf, stage_lo, 0, _H, 0, 15)
            if p == 0:
                _issue_lo(x_hbm, stage_lo, sem_lo, b + 1)
            else:
                @pl.when(t < _BPW // 2 - 1)
                def _():
                    _issue_lo(x_hbm, stage_lo, sem_lo, b + 1)
            pltpu.async_copy(obuf, out_hbm.at[b], osem)
        return carry

    lax.fori_loop(0, _BPW // 2, step, 0)
    pltpu.make_async_copy(out0, out_hbm.at[0], sem_o0).wait()
    pltpu.make_async_copy(out1, out_hbm.at[0], sem_o1).wait()


@jax.jit
def _run(x):
    f = pl.kernel(
        _body,
        out_type=jax.ShapeDtypeStruct((_B, _OUT), jnp.float32),
        mesh=plsc.VectorSubcoreMesh(core_axis_name="c", subcore_axis_name="s"),
        scratch_types=[
            pltpu.VMEM((_H, _N), jnp.float32),
            pltpu.VMEM((_H, _H), jnp.float32),
            pltpu.VMEM((_OUT,), jnp.float32),
            pltpu.VMEM((_OUT,), jnp.float32),
            pltpu.SemaphoreType.DMA,
            pltpu.SemaphoreType.DMA,
            pltpu.SemaphoreType.DMA,
            pltpu.SemaphoreType.DMA,
        ],
    )
    return f(x)


def kernel(input):
    return _run(input)


# speedup vs baseline: 1.5209x; 1.0827x over previous
import jax
import jax.numpy as jnp
from jax import lax
from jax.experimental import pallas as pl
from jax.experimental.pallas import tpu as pltpu
from jax.experimental.pallas import tpu_sc as plsc

_N = 256
_H = 128
_B = 1024
_OUT = _N * (_N + 1) // 2
_NC = 2
_NS = 16
_NW = _NC * _NS
_BPW = _B // _NW

_OFF = [i * _N - (i * (i - 1)) // 2 for i in range(_N)]


def _phase1_heads(obuf, stage, lo, hi, col0):
    for i in range(hi - 1, lo - 1, -1):
        u = i & 15
        obuf[pl.ds(_OFF[i] - u, 16)] = stage[i - lo, pl.ds(i - u - col0, 16)]


def _phase2_interior(obuf, stage, lo, hi, col0, kmax):
    for k in range(1, kmax + 1):
        i1 = min(hi, _N - 16 * k)
        if i1 <= lo:
            continue
        col = _N - 16 * k
        scol = col - col0
        dst0 = jnp.int32(_OFF[lo] + col - lo)
        delta0 = jnp.int32(_N - 1 - lo)

        @plsc.parallel_loop(lo, i1, unroll=16, carry=(dst0, delta0))
        def _(i, c):
            dst, delta = c
            obuf[pl.ds(dst, 16)] = stage[i - lo, pl.ds(scol, 16)]
            return (dst + delta, delta - 1)


_PIECES = ((_H, _N, _H), (64, _H, 0), (0, 64, 0))


def _issue_piece(x_hbm, stage, sem, b, piece):
    lo, hi, col0 = piece
    pltpu.async_copy(
        x_hbm.at[b, pl.ds(lo, hi - lo), pl.ds(col0, _N - col0)], stage, sem)


def _body(x_hbm, out_hbm, stage0, stage1, stage2, out0, out1,
          sem_s0, sem_s1, sem_s2, sem_o0, sem_o1):
    wid = lax.axis_index("s") * _NC + lax.axis_index("c")
    stages = (stage0, stage1, stage2)
    ssems = (sem_s0, sem_s1, sem_s2)
    outbufs = (out0, out1)
    osems = (sem_o0, sem_o1)
    first = wid * _BPW
    for s in range(3):
        _issue_piece(x_hbm, stages[s], ssems[s], first, _PIECES[s])

    def step(t, carry):
        for p in range(2):
            b = first + 2 * t + p
            obuf, osem = outbufs[p], osems[p]
            for s in range(3):
                lo, hi, col0 = _PIECES[s]
                pltpu.make_async_copy(
                    x_hbm.at[b, pl.ds(lo, hi - lo), pl.ds(col0, _N - col0)],
                    stages[s], ssems[s]).wait()
                if s == 0:
                    @pl.when(t > 0)
                    def _():
                        pltpu.make_async_copy(
                            obuf, out_hbm.at[b], osem).wait()
                _phase1_heads(obuf, stages[s], lo, hi, col0)
                _phase2_interior(obuf, stages[s], lo, hi, col0, 15)
                if p == 0:
                    _issue_piece(x_hbm, stages[s], ssems[s], b + 1,
                                 _PIECES[s])
                else:
                    @pl.when(t < _BPW // 2 - 1)
                    def _():
                        _issue_piece(x_hbm, stages[s], ssems[s], b + 1,
                                     _PIECES[s])
            pltpu.async_copy(obuf, out_hbm.at[b], osem)
        return carry

    lax.fori_loop(0, _BPW // 2, step, 0)
    pltpu.make_async_copy(out0, out_hbm.at[0], sem_o0).wait()
    pltpu.make_async_copy(out1, out_hbm.at[0], sem_o1).wait()


@jax.jit
def _run(x):
    f = pl.kernel(
        _body,
        out_type=jax.ShapeDtypeStruct((_B, _OUT), jnp.float32),
        mesh=plsc.VectorSubcoreMesh(core_axis_name="c", subcore_axis_name="s"),
        scratch_types=[
            pltpu.VMEM((_PIECES[s][1] - _PIECES[s][0],
                        _N - _PIECES[s][2]), jnp.float32)
            for s in range(3)
        ] + [
            pltpu.VMEM((_OUT,), jnp.float32),
            pltpu.VMEM((_OUT,), jnp.float32),
            pltpu.SemaphoreType.DMA,
            pltpu.SemaphoreType.DMA,
            pltpu.SemaphoreType.DMA,
            pltpu.SemaphoreType.DMA,
            pltpu.SemaphoreType.DMA,
        ],
    )
    return f(x)


def kernel(input):
    return _run(input)
